# baseline (device time: 18347 ns/iter reference)
import jax
import jax.numpy as jnp
from jax import lax
from jax.experimental import pallas as pl
from jax.experimental.pallas import tpu as pltpu


def kernel(x):
    _, m, n2 = x.shape
    n = n2 // 2
    qr = m // 4
    hr = qr // 2

    def body(x_ref, out_ref, rbuf, send_sems, recv_sems):
        my_x = lax.axis_index("x")
        my_y = lax.axis_index("y")
        my_z = lax.axis_index("z")
        partner = (1 - my_x, my_y, my_z)
        ynbr = (my_x, 1 - my_y, my_z)
        znbr = (my_x, my_y, 1 - my_z)

        q0 = 2 * my_y + my_z
        q1 = 2 * (1 - my_y) + my_z
        q2 = 2 * my_y + (1 - my_z)
        q3 = 2 * (1 - my_y) + (1 - my_z)

        barrier_sem = pltpu.get_barrier_semaphore()
        for nbr in (partner, ynbr, znbr):
            pl.semaphore_signal(
                barrier_sem, inc=1,
                device_id=nbr, device_id_type=pl.DeviceIdType.MESH,
            )
        pl.semaphore_wait(barrier_sem, 3)

        send_col = (1 - my_x) * n
        keep_col = my_x * n

        def rc(src, dst, i, dev):
            return pltpu.make_async_remote_copy(
                src_ref=src, dst_ref=dst,
                send_sem=send_sems.at[i], recv_sem=recv_sems.at[i],
                device_id=dev, device_id_type=pl.DeviceIdType.MESH,
            )

        s1 = rc(
            x_ref.at[0, pl.ds(q0 * qr, qr), pl.ds(send_col, n)],
            rbuf.at[q0], 0, partner,
        )
        s1.start()

        rc(rbuf.at[q0], rbuf.at[q0], 0, partner).wait_recv()

        s2 = rc(rbuf.at[q0], rbuf.at[q0], 1, ynbr)
        s2.start()
        s3 = rc(rbuf.at[q0], rbuf.at[q0], 2, znbr)
        s3.start()

        rc(rbuf.at[q1], rbuf.at[q1], 1, ynbr).wait_recv()
        s4 = rc(
            rbuf.at[q1, pl.ds(0, hr), :], rbuf.at[q1, pl.ds(0, hr), :],
            4, znbr,
        )
        s4.start()

        rc(rbuf.at[q2], rbuf.at[q2], 2, znbr).wait_recv()
        s5 = rc(
            rbuf.at[q2, pl.ds(hr, hr), :], rbuf.at[q2, pl.ds(hr, hr), :],
            3, ynbr,
        )
        s5.start()

        rc(rbuf.at[q3, pl.ds(hr, hr), :], rbuf.at[q3, pl.ds(hr, hr), :],
           3, ynbr).wait_recv()
        rc(rbuf.at[q3, pl.ds(0, hr), :], rbuf.at[q3, pl.ds(0, hr), :],
           4, znbr).wait_recv()

        out_ref[:, :] = (
            x_ref[0, :, pl.ds(keep_col, n)] + rbuf[:, :, :].reshape(m, n)
        )

        for s in (s1, s2, s3, s4, s5):
            s.wait_send()

    return pl.pallas_call(
        body,
        out_shape=jax.ShapeDtypeStruct((m, n), x.dtype),
        in_specs=[pl.BlockSpec(memory_space=pltpu.VMEM)],
        out_specs=pl.BlockSpec(memory_space=pltpu.VMEM),
        scratch_shapes=[
            pltpu.VMEM((4, qr, n), x.dtype),
            pltpu.SemaphoreType.DMA((5,)),
            pltpu.SemaphoreType.DMA((5,)),
        ],
        compiler_params=pltpu.CompilerParams(collective_id=0),
    )(x)


# device time: 17226 ns/iter; 1.0651x vs baseline; 1.0651x over previous
import jax
import jax.numpy as jnp
from jax import lax
from jax.experimental import pallas as pl
from jax.experimental.pallas import tpu as pltpu

NPIECE = 2


def kernel(x):
    _, m, n2 = x.shape
    n = n2 // 2
    qr = m // 4
    pr = qr // NPIECE

    def body(x_ref, out_ref, rbuf, send_sems, recv_sems):
        my_x = lax.axis_index("x")
        my_y = lax.axis_index("y")
        my_z = lax.axis_index("z")
        partner = (1 - my_x, my_y, my_z)
        ynbr = (my_x, 1 - my_y, my_z)
        znbr = (my_x, my_y, 1 - my_z)
        diag = (my_x, 1 - my_y, 1 - my_z)

        q0 = 2 * my_y + my_z
        q1 = 2 * (1 - my_y) + my_z
        q2 = 2 * my_y + (1 - my_z)
        q3 = 2 * (1 - my_y) + (1 - my_z)

        barrier_sem = pltpu.get_barrier_semaphore()
        for nbr in (partner, ynbr, znbr, diag):
            pl.semaphore_signal(
                barrier_sem, inc=1,
                device_id=nbr, device_id_type=pl.DeviceIdType.MESH,
            )
        pl.semaphore_wait(barrier_sem, 4)

        send_col = (1 - my_x) * n
        keep_col = my_x * n

        def rc(src, dst, i, dev):
            return pltpu.make_async_remote_copy(
                src_ref=src, dst_ref=dst,
                send_sem=send_sems.at[i], recv_sem=recv_sems.at[i],
                device_id=dev, device_id_type=pl.DeviceIdType.MESH,
            )

        sends = []
        for p in range(NPIECE):
            s = rc(
                x_ref.at[0, pl.ds(q0 * qr + p * pr, pr), pl.ds(send_col, n)],
                rbuf.at[q0, pl.ds(p * pr, pr), :], p, partner,
            )
            s.start()
            sends.append(s)

        for p in range(NPIECE):
            piece = rbuf.at[q0, pl.ds(p * pr, pr), :]
            rc(piece, piece, p, partner).wait_recv()
            for base, dev in ((2, ynbr), (4, znbr), (6, diag)):
                s = rc(piece, rbuf.at[q0, pl.ds(p * pr, pr), :],
                       base + p, dev)
                s.start()
                sends.append(s)

        for qk, base in ((q1, 2), (q2, 4), (q3, 6)):
            for p in range(NPIECE):
                sl = rbuf.at[qk, pl.ds(p * pr, pr), :]
                rc(sl, sl, base + p, ynbr).wait_recv()

        out_ref[:, :] = (
            x_ref[0, :, pl.ds(keep_col, n)] + rbuf[:, :, :].reshape(m, n)
        )

        for s in sends:
            s.wait_send()

    return pl.pallas_call(
        body,
        out_shape=jax.ShapeDtypeStruct((m, n), x.dtype),
        in_specs=[pl.BlockSpec(memory_space=pltpu.VMEM)],
        out_specs=pl.BlockSpec(memory_space=pltpu.VMEM),
        scratch_shapes=[
            pltpu.VMEM((4, qr, n), x.dtype),
            pltpu.SemaphoreType.DMA((8,)),
            pltpu.SemaphoreType.DMA((8,)),
        ],
        compiler_params=pltpu.CompilerParams(collective_id=0),
    )(x)


# device time: 15419 ns/iter; 1.1899x vs baseline; 1.1172x over previous
import jax
import jax.numpy as jnp
from jax import lax
from jax.experimental import pallas as pl
from jax.experimental.pallas import tpu as pltpu

Q = 80


def kernel(x):
    _, m, n2 = x.shape
    n = n2 // 2
    d = m - 4 * Q

    def body(x_ref, out_ref, rbuf, send_sems, recv_sems):
        my_x = lax.axis_index("x")
        my_y = lax.axis_index("y")
        my_z = lax.axis_index("z")
        partner = (1 - my_x, my_y, my_z)
        ynbr = (my_x, 1 - my_y, my_z)
        znbr = (my_x, my_y, 1 - my_z)
        diag = (my_x, 1 - my_y, 1 - my_z)

        q0 = 2 * my_y + my_z
        q1 = 2 * (1 - my_y) + my_z
        q2 = 2 * my_y + (1 - my_z)
        q3 = 2 * (1 - my_y) + (1 - my_z)

        barrier_sem = pltpu.get_barrier_semaphore()
        for nbr in (partner, ynbr, znbr, diag):
            pl.semaphore_signal(
                barrier_sem, inc=1,
                device_id=nbr, device_id_type=pl.DeviceIdType.MESH,
            )
        pl.semaphore_wait(barrier_sem, 4)

        send_col = (1 - my_x) * n
        keep_col = my_x * n

        def rc(src, dst, i, dev):
            return pltpu.make_async_remote_copy(
                src_ref=src, dst_ref=dst,
                send_sem=send_sems.at[i], recv_sem=recv_sems.at[i],
                device_id=dev, device_id_type=pl.DeviceIdType.MESH,
            )

        s_xq = rc(
            x_ref.at[0, pl.ds(q0 * Q, Q), pl.ds(send_col, n)],
            rbuf.at[pl.ds(q0 * Q, Q), :], 1, partner,
        )
        s_xq.start()
        s_dir = rc(
            x_ref.at[0, pl.ds(4 * Q, d), pl.ds(send_col, n)],
            rbuf.at[pl.ds(4 * Q, d), :], 0, partner,
        )
        s_dir.start()

        myq = rbuf.at[pl.ds(q0 * Q, Q), :]
        rc(myq, myq, 1, partner).wait_recv()
        s_y = rc(myq, myq, 2, ynbr)
        s_y.start()
        s_z = rc(myq, myq, 3, znbr)
        s_z.start()
        s_d = rc(myq, myq, 4, diag)
        s_d.start()

        for qk, i in ((q1, 2), (q2, 3), (q3, 4)):
            sl = rbuf.at[pl.ds(qk * Q, Q), :]
            rc(sl, sl, i, ynbr).wait_recv()
        dl = rbuf.at[pl.ds(4 * Q, d), :]
        rc(dl, dl, 0, partner).wait_recv()

        out_ref[:, :] = x_ref[0, :, pl.ds(keep_col, n)] + rbuf[:, :]

        for s in (s_xq, s_dir, s_y, s_z, s_d):
            s.wait_send()

    return pl.pallas_call(
        body,
        out_shape=jax.ShapeDtypeStruct((m, n), x.dtype),
        in_specs=[pl.BlockSpec(memory_space=pltpu.VMEM)],
        out_specs=pl.BlockSpec(memory_space=pltpu.VMEM),
        scratch_shapes=[
            pltpu.VMEM((m, n), x.dtype),
            pltpu.SemaphoreType.DMA((5,)),
            pltpu.SemaphoreType.DMA((5,)),
        ],
        compiler_params=pltpu.CompilerParams(collective_id=0),
    )(x)


# device time: 13795 ns/iter; 1.3300x vs baseline; 1.1177x over previous
import jax
import jax.numpy as jnp
from jax import lax
from jax.experimental import pallas as pl
from jax.experimental.pallas import tpu as pltpu

Q = 128
NP = 4
PR = Q // NP


def kernel(x):
    _, m, n2 = x.shape
    n = n2 // 2

    def body(x_ref, out_ref, rbuf, send_sems, recv_sems):
        my_x = lax.axis_index("x")
        my_y = lax.axis_index("y")
        my_z = lax.axis_index("z")
        partner = (1 - my_x, my_y, my_z)
        ynbr = (my_x, 1 - my_y, my_z)
        znbr = (my_x, my_y, 1 - my_z)

        q0 = 2 * my_y + my_z
        q1 = 2 * (1 - my_y) + my_z
        q2 = 2 * my_y + (1 - my_z)
        q3 = 2 * (1 - my_y) + (1 - my_z)

        barrier_sem = pltpu.get_barrier_semaphore()
        for nbr in (partner, ynbr, znbr):
            pl.semaphore_signal(
                barrier_sem, inc=1,
                device_id=nbr, device_id_type=pl.DeviceIdType.MESH,
            )
        pl.semaphore_wait(barrier_sem, 3)

        send_col = (1 - my_x) * n
        keep_col = my_x * n

        def rc(src, dst, i, dev):
            return pltpu.make_async_remote_copy(
                src_ref=src, dst_ref=dst,
                send_sem=send_sems.at[i], recv_sem=recv_sems.at[i],
                device_id=dev, device_id_type=pl.DeviceIdType.MESH,
            )

        yb = NP + 1
        zb = yb + NP

        sends = []
        for p in range(NP):
            s = rc(
                x_ref.at[0, pl.ds(q0 * Q + p * PR, PR), pl.ds(send_col, n)],
                rbuf.at[pl.ds(q0 * Q + p * PR, PR), :], p, partner,
            )
            s.start()
            sends.append(s)
        s = rc(
            x_ref.at[0, pl.ds(q3 * Q, Q), pl.ds(send_col, n)],
            rbuf.at[pl.ds(q3 * Q, Q), :], NP, partner,
        )
        s.start()
        sends.append(s)

        for p in range(NP):
            piece = rbuf.at[pl.ds(q0 * Q + p * PR, PR), :]
            rc(piece, piece, p, partner).wait_recv()
            for i, dev in ((yb + p, ynbr), (zb + p, znbr)):
                s = rc(piece, piece, i, dev)
                s.start()
                sends.append(s)

        for qk, base in ((q1, yb), (q2, zb)):
            for p in range(NP):
                sl = rbuf.at[pl.ds(qk * Q + p * PR, PR), :]
                rc(sl, sl, base + p, ynbr).wait_recv()
        sl = rbuf.at[pl.ds(q3 * Q, Q), :]
        rc(sl, sl, NP, partner).wait_recv()

        out_ref[:, :] = x_ref[0, :, pl.ds(keep_col, n)] + rbuf[:, :]

        for s in sends:
            s.wait_send()

    nsem = NP + 1 + 2 * NP
    return pl.pallas_call(
        body,
        out_shape=jax.ShapeDtypeStruct((m, n), x.dtype),
        in_specs=[pl.BlockSpec(memory_space=pltpu.VMEM)],
        out_specs=pl.BlockSpec(memory_space=pltpu.VMEM),
        scratch_shapes=[
            pltpu.VMEM((m, n), x.dtype),
            pltpu.SemaphoreType.DMA((nsem,)),
            pltpu.SemaphoreType.DMA((nsem,)),
        ],
        compiler_params=pltpu.CompilerParams(collective_id=0),
    )(x)
